# baseline (speedup 1.0000x reference)
import functools

import jax
import jax.numpy as jnp
from jax import lax
from jax.experimental import pallas as pl
from jax.experimental.pallas import tpu as pltpu
from jax.experimental.pallas import tpu_sc as plsc

N = 10000
E = 320000
D = 128

NC = 2
NS = 16
NW = NC * NS
EPW = E // NW
CHUNK = 40
NCHUNK = EPW // CHUNK
RCH = 40
NRCH = N // RCH
LANES = 16


def _sc_aggregate(embeds_bf, adj_flat, dst_flat, src_flat):
    mesh = plsc.VectorSubcoreMesh(core_axis_name="c", subcore_axis_name="s")

    @functools.partial(
        pl.kernel,
        mesh=mesh,
        out_type=jax.ShapeDtypeStruct((NC, N, D), jnp.float32),
        compiler_params=pltpu.CompilerParams(needs_layout_passes=False,
                                             use_tc_tiling_on_sc=False),
        scratch_types=(
            [pltpu.VMEM((EPW,), jnp.int32)]
            + [pltpu.VMEM((CHUNK,), jnp.float32) for _ in range(3)]
            + [pltpu.VMEM((CHUNK,), jnp.int32) for _ in range(3)]
            + [pltpu.VMEM((CHUNK, D // 2), jnp.int32) for _ in range(3)]
            + [pltpu.VMEM((CHUNK, D), jnp.float32) for _ in range(3)]
            + [pltpu.VMEM_SHARED((N, D), jnp.float32)]
            + [pltpu.SemaphoreType.DMA for _ in range(9)]
        ),
    )
    def body(embeds_hbm, adj_hbm, dst_hbm, src_hbm, out_hbm, *refs):
        src_v = refs[0]
        abufs = refs[1:4]
        dbufs = refs[4:7]
        gbufs = refs[7:10]
        fbufs = refs[10:13]
        acc_sh = refs[13]
        isems = refs[14:17]
        gsems = refs[17:20]
        ssems = refs[20:23]

        cid = lax.axis_index("c")
        sid = lax.axis_index("s")
        wid = cid * NS + sid

        zero16 = jnp.zeros((LANES,), jnp.float32)
        for i in range(CHUNK):
            for j in range(D // LANES):
                fbufs[0][i, pl.ds(LANES * j, LANES)] = zero16
        for k in range((NRCH + NS - 1) // NS):
            rc = sid + NS * k
            @pl.when(rc < NRCH)
            def _():
                pltpu.sync_copy(fbufs[0], acc_sh.at[pl.ds(rc * RCH, RCH)])
        plsc.subcore_barrier()

        base = wid * EPW

        pltpu.sync_copy(src_hbm.at[pl.ds(base, EPW)], src_v)

        def icopies(ci, b):
            return (
                pltpu.make_async_copy(
                    adj_hbm.at[pl.ds(base + ci * CHUNK, CHUNK)], abufs[b],
                    isems[b]),
                pltpu.make_async_copy(
                    dst_hbm.at[pl.ds(base + ci * CHUNK, CHUNK)], dbufs[b],
                    isems[b]),
            )

        def i_start(ci, b):
            for c in icopies(ci, b):
                c.start()

        def i_wait(ci, b):
            for c in icopies(ci, b):
                c.wait()

        def gcopy(ci, b):
            idx = src_v.at[pl.ds(ci * CHUNK, CHUNK)]
            return pltpu.make_async_copy(
                embeds_hbm.at[idx], gbufs[b], gsems[b])

        def scopy_start(b):
            pltpu.async_copy(fbufs[b], acc_sh.at[dbufs[b]], ssems[b],
                             add=True)

        def scopy_wait(b):
            pltpu.make_async_copy(fbufs[b], acc_sh.at[dbufs[b]],
                                  ssems[b]).wait()

        def scale(ci, b):
            gb = gbufs[b]
            fb = fbufs[b]
            ab = abufs[b]
            for g in range((CHUNK + LANES - 1) // LANES):
                off = min(g * LANES, CHUNK - LANES)
                a16 = ab[pl.ds(off, LANES)]
                lo = g * LANES
                hi = min(lo + LANES, CHUNK)
                for e in range(lo, hi):
                    av = jnp.full((LANES,), a16[e - off], jnp.float32)
                    for j in range(D // (2 * LANES)):
                        v16i = gb[e, pl.ds(LANES * j, LANES)]
                        v32 = plsc.bitcast(v16i, jnp.bfloat16)
                        lo_f, hi_f = plsc.unpack(
                            v32, format=plsc.PackFormat.INTERLEAVED)
                        fb[e, pl.ds(2 * LANES * j, LANES)] = lo_f * av
                        fb[e, pl.ds(2 * LANES * j + LANES, LANES)] = hi_f * av

        i_start(0, 0)
        i_start(1, 1)
        gcopy(0, 0).start()
        gcopy(1, 1).start()

        NITER = (NCHUNK + 2) // 3

        def iter_body(i, carry):
            for u in range(3):
                c = 3 * i + u

                @pl.when(c < NCHUNK)
                def _():
                    bn = (u + 2) % 3
                    bp = (u - 1) % 3

                    gcopy(c, u).wait()

                    @pl.when(c + 2 < NCHUNK)
                    def _():
                        gcopy(c + 2, bn).start()

                    i_wait(c, u)
                    scale(c, u)
                    scopy_start(u)

                    @pl.when(c >= 1)
                    def _():
                        scopy_wait(bp)

                    @pl.when(c + 2 < NCHUNK)
                    def _():
                        i_start(c + 2, bp)

            return carry

        lax.fori_loop(0, NITER, iter_body, 0)
        scopy_wait((NCHUNK - 1) % 3)

        plsc.subcore_barrier()
        for k in range((NRCH + NS - 1) // NS):
            rc = sid + NS * k
            @pl.when(rc < NRCH)
            def _():
                pltpu.sync_copy(acc_sh.at[pl.ds(rc * RCH, RCH)],
                                out_hbm.at[cid, pl.ds(rc * RCH, RCH)])

    return body(embeds_bf, adj_flat, dst_flat, src_flat)


def _tc_combine(p0, p1, W):
    BLK = 1000

    def body(p0_ref, p1_ref, w_ref, o_ref):
        x = p0_ref[...] + p1_ref[...]
        y = lax.dot_general(x, w_ref[...], (((1,), (1,)), ((), ())),
                            preferred_element_type=jnp.float32)
        o_ref[...] = jnp.where(y >= 0, y, 0.2 * y)

    return pl.pallas_call(
        body,
        grid=(N // BLK,),
        in_specs=[
            pl.BlockSpec((BLK, D), lambda i: (i, 0)),
            pl.BlockSpec((BLK, D), lambda i: (i, 0)),
            pl.BlockSpec((D, D), lambda i: (0, 0)),
        ],
        out_specs=pl.BlockSpec((BLK, D), lambda i: (i, 0)),
        out_shape=jax.ShapeDtypeStruct((N, D), jnp.float32),
    )(p0, p1, W)


def kernel(embeds, adj_values, edge_index, W):
    dst = edge_index[0].astype(jnp.int32)
    src = edge_index[1].astype(jnp.int32)
    embeds_bf = (embeds.reshape(N, D // 32, 2, 16)
                 .transpose(0, 1, 3, 2)
                 .reshape(N, D // 2, 2)
                 .astype(jnp.bfloat16))
    embeds_bf = lax.bitcast_convert_type(embeds_bf, jnp.int32)
    partials = _sc_aggregate(embeds_bf, adj_values, dst, src)
    return _tc_combine(partials[0], partials[1], W)


# speedup vs baseline: 1.0274x; 1.0274x over previous
import functools

import jax
import jax.numpy as jnp
from jax import lax
from jax.experimental import pallas as pl
from jax.experimental.pallas import tpu as pltpu
from jax.experimental.pallas import tpu_sc as plsc

N = 10000
E = 320000
D = 128

NC = 2
NS = 16
NW = NC * NS
EPW = E // NW
CHUNK = 40
NCHUNK = EPW // CHUNK
RCH = 40
NRCH = N // RCH
LANES = 16


def _sc_aggregate(embeds_bf, adj_flat, dst_flat, src_flat):
    mesh = plsc.VectorSubcoreMesh(core_axis_name="c", subcore_axis_name="s")

    @functools.partial(
        pl.kernel,
        mesh=mesh,
        out_type=jax.ShapeDtypeStruct((NC, N, D), jnp.float32),
        compiler_params=pltpu.CompilerParams(needs_layout_passes=False,
                                             use_tc_tiling_on_sc=False),
        scratch_types=(
            [pltpu.VMEM((EPW,), jnp.int32)]
            + [pltpu.VMEM((CHUNK,), jnp.float32) for _ in range(3)]
            + [pltpu.VMEM((CHUNK,), jnp.int32) for _ in range(3)]
            + [pltpu.VMEM((CHUNK, D // 2), jnp.int32) for _ in range(3)]
            + [pltpu.VMEM((CHUNK, D), jnp.float32) for _ in range(3)]
            + [pltpu.VMEM_SHARED((N, D), jnp.float32)]
            + [pltpu.SemaphoreType.DMA for _ in range(9)]
        ),
    )
    def body(embeds_hbm, adj_hbm, dst_hbm, src_hbm, out_hbm, *refs):
        src_v = refs[0]
        abufs = refs[1:4]
        dbufs = refs[4:7]
        gbufs = refs[7:10]
        fbufs = refs[10:13]
        acc_sh = refs[13]
        isems = refs[14:17]
        gsems = refs[17:20]
        ssems = refs[20:23]

        cid = lax.axis_index("c")
        sid = lax.axis_index("s")
        wid = cid * NS + sid

        zero16 = jnp.zeros((LANES,), jnp.float32)
        for i in range(CHUNK):
            for j in range(D // LANES):
                fbufs[0][i, pl.ds(LANES * j, LANES)] = zero16
        for k in range((NRCH + NS - 1) // NS):
            rc = sid + NS * k
            @pl.when(rc < NRCH)
            def _():
                pltpu.make_async_copy(
                    fbufs[0], acc_sh.at[pl.ds(rc * RCH, RCH)],
                    ssems[0]).start()
        for k in range((NRCH + NS - 1) // NS):
            rc = sid + NS * k
            @pl.when(rc < NRCH)
            def _():
                pltpu.make_async_copy(
                    fbufs[0], acc_sh.at[pl.ds(rc * RCH, RCH)],
                    ssems[0]).wait()
        plsc.subcore_barrier()

        base = wid * EPW

        pltpu.sync_copy(src_hbm.at[pl.ds(base, EPW)], src_v)

        def icopies(ci, b):
            return (
                pltpu.make_async_copy(
                    adj_hbm.at[pl.ds(base + ci * CHUNK, CHUNK)], abufs[b],
                    isems[b]),
                pltpu.make_async_copy(
                    dst_hbm.at[pl.ds(base + ci * CHUNK, CHUNK)], dbufs[b],
                    isems[b]),
            )

        def i_start(ci, b):
            for c in icopies(ci, b):
                c.start()

        def i_wait(ci, b):
            for c in icopies(ci, b):
                c.wait()

        def gcopy(ci, b):
            idx = src_v.at[pl.ds(ci * CHUNK, CHUNK)]
            return pltpu.make_async_copy(
                embeds_hbm.at[idx], gbufs[b], gsems[b])

        def scopy_start(b):
            pltpu.async_copy(fbufs[b], acc_sh.at[dbufs[b]], ssems[b],
                             add=True)

        def scopy_wait(b):
            pltpu.make_async_copy(fbufs[b], acc_sh.at[dbufs[b]],
                                  ssems[b]).wait()

        def scale(ci, b):
            gb = gbufs[b]
            fb = fbufs[b]
            ab = abufs[b]
            for g in range((CHUNK + LANES - 1) // LANES):
                off = min(g * LANES, CHUNK - LANES)
                a16 = ab[pl.ds(off, LANES)]
                lo = g * LANES
                hi = min(lo + LANES, CHUNK)
                for e in range(lo, hi):
                    av = jnp.full((LANES,), a16[e - off], jnp.float32)
                    for j in range(D // (2 * LANES)):
                        v16i = gb[e, pl.ds(LANES * j, LANES)]
                        v32 = plsc.bitcast(v16i, jnp.bfloat16)
                        lo_f, hi_f = plsc.unpack(
                            v32, format=plsc.PackFormat.INTERLEAVED)
                        fb[e, pl.ds(2 * LANES * j, LANES)] = lo_f * av
                        fb[e, pl.ds(2 * LANES * j + LANES, LANES)] = hi_f * av

        i_start(0, 0)
        i_start(1, 1)
        gcopy(0, 0).start()
        gcopy(1, 1).start()

        NITER = (NCHUNK + 2) // 3

        def iter_body(i, carry):
            for u in range(3):
                c = 3 * i + u

                @pl.when(c < NCHUNK)
                def _():
                    bn = (u + 2) % 3
                    bp = (u - 1) % 3

                    gcopy(c, u).wait()

                    @pl.when(c + 2 < NCHUNK)
                    def _():
                        gcopy(c + 2, bn).start()

                    i_wait(c, u)
                    scale(c, u)
                    scopy_start(u)

                    @pl.when(c >= 1)
                    def _():
                        scopy_wait(bp)

                    @pl.when(c + 2 < NCHUNK)
                    def _():
                        i_start(c + 2, bp)

            return carry

        lax.fori_loop(0, NITER, iter_body, 0)
        scopy_wait((NCHUNK - 1) % 3)

        plsc.subcore_barrier()
        for k in range((NRCH + NS - 1) // NS):
            rc = sid + NS * k
            @pl.when(rc < NRCH)
            def _():
                pltpu.make_async_copy(
                    acc_sh.at[pl.ds(rc * RCH, RCH)],
                    out_hbm.at[cid, pl.ds(rc * RCH, RCH)], ssems[0]).start()
        for k in range((NRCH + NS - 1) // NS):
            rc = sid + NS * k
            @pl.when(rc < NRCH)
            def _():
                pltpu.make_async_copy(
                    acc_sh.at[pl.ds(rc * RCH, RCH)],
                    out_hbm.at[cid, pl.ds(rc * RCH, RCH)], ssems[0]).wait()

    return body(embeds_bf, adj_flat, dst_flat, src_flat)


def _tc_combine(p0, p1, W):
    BLK = 1000

    def body(p0_ref, p1_ref, w_ref, o_ref):
        x = p0_ref[...] + p1_ref[...]
        y = lax.dot_general(x, w_ref[...], (((1,), (1,)), ((), ())),
                            preferred_element_type=jnp.float32)
        o_ref[...] = jnp.where(y >= 0, y, 0.2 * y)

    return pl.pallas_call(
        body,
        grid=(N // BLK,),
        in_specs=[
            pl.BlockSpec((BLK, D), lambda i: (i, 0)),
            pl.BlockSpec((BLK, D), lambda i: (i, 0)),
            pl.BlockSpec((D, D), lambda i: (0, 0)),
        ],
        out_specs=pl.BlockSpec((BLK, D), lambda i: (i, 0)),
        out_shape=jax.ShapeDtypeStruct((N, D), jnp.float32),
    )(p0, p1, W)


def kernel(embeds, adj_values, edge_index, W):
    dst = edge_index[0].astype(jnp.int32)
    src = edge_index[1].astype(jnp.int32)
    embeds_bf = (embeds.reshape(N, D // 32, 2, 16)
                 .transpose(0, 1, 3, 2)
                 .reshape(N, D // 2, 2)
                 .astype(jnp.bfloat16))
    embeds_bf = lax.bitcast_convert_type(embeds_bf, jnp.int32)
    partials = _sc_aggregate(embeds_bf, adj_values, dst, src)
    return _tc_combine(partials[0], partials[1], W)
